# baseline (speedup 1.0000x reference)
import functools

import jax
import jax.numpy as jnp
from jax import lax
from jax.experimental import pallas as pl
from jax.experimental.pallas import tpu as pltpu
from jax.experimental.pallas import tpu_sc as plsc

N_NODES = 50000
RT = 512
NRT = 98
NP = RT * NRT
N_EDGES = 800000
ESTEP = 256
NSTEP = 98
GSTEP = 14
EP = 32 * NSTEP * ESTEP
RSUB = NP // 16
N_GRAPHS = 256
SEQ = 1000


@functools.cache
def _sc_segsum(C):
    mesh = plsc.VectorSubcoreMesh(core_axis_name="c", subcore_axis_name="s")

    def body(*args):
        vals = args[:C]
        srcs, dsts, zeros, out = args[C:C + 4]
        src_idx, dst_idx, rows, acc, gsem, ssem = args[C + 4:]
        cid = lax.axis_index("c")
        sid = lax.axis_index("s")
        w = cid * 16 + sid

        def gather(c, j, buf):
            pltpu.async_copy(vals[c].at[src_idx.at[j]], rows.at[buf],
                             gsem.at[buf])

        def scat_desc(j, buf):
            return pltpu.make_async_copy(rows.at[buf], acc.at[dst_idx.at[j]],
                                         ssem.at[buf])

        for c in range(C):
            pltpu.sync_copy(zeros.at[pl.ds(sid * RSUB, RSUB)],
                            acc.at[pl.ds(sid * RSUB, RSUB)])
            plsc.subcore_barrier()

            for g in range(NSTEP // GSTEP):
                pltpu.sync_copy(srcs.at[w].at[pl.ds(g * GSTEP, GSTEP)], src_idx)
                pltpu.sync_copy(dsts.at[w].at[pl.ds(g * GSTEP, GSTEP)], dst_idx)
                gather(c, 0, 0)

                def step(j, carry):
                    buf = j % 2
                    pltpu.make_async_copy(vals[c].at[src_idx.at[j]],
                                          rows.at[buf], gsem.at[buf]).wait()

                    @pl.when(j >= 1)
                    def _():
                        scat_desc(j, 1 - buf).wait()

                    @pl.when(j + 1 < GSTEP)
                    def _():
                        gather(c, j + 1, 1 - buf)

                    scat_desc(j, buf).start(add=True)
                    return carry

                lax.fori_loop(0, GSTEP, step, 0)
                scat_desc(0, (GSTEP - 1) % 2).wait()
            plsc.subcore_barrier()
            pltpu.sync_copy(acc.at[pl.ds(sid * RSUB, RSUB)],
                            out.at[cid].at[c].at[pl.ds(sid * RSUB, RSUB)])
            if c + 1 < C:
                plsc.subcore_barrier()

    return pl.kernel(
        body,
        out_type=jax.ShapeDtypeStruct((2, C, NP, 32), jnp.float32),
        mesh=mesh,
        scratch_types=[
            pltpu.VMEM((GSTEP, ESTEP), jnp.int32),
            pltpu.VMEM((GSTEP, ESTEP), jnp.int32),
            pltpu.VMEM((2, ESTEP, 32), jnp.float32),
            pltpu.VMEM_SHARED((NP, 32), jnp.float32),
            pltpu.SemaphoreType.DMA((2,)),
            pltpu.SemaphoreType.DMA((2,)),
        ],
        compiler_params=pltpu.CompilerParams(use_tc_tiling_on_sc=False),
    )


def _dense_raw(P_list, U_list, Wr_list, Wt_list, dout):
    nP, nU = len(P_list), len(U_list)

    def body(*args):
        i = pl.program_id(0)
        it = iter(args)
        Ps = [next(it) for _ in range(nP)]
        Wrs = [next(it) for _ in range(nP)]
        Us = [next(it) for _ in range(nU)]
        Wts = [next(it) for _ in range(nU)]
        h_ref = next(it)
        st_ref = next(it)
        acc = next(it)
        h = jnp.zeros((RT, dout), jnp.float32)
        for p, w in zip(Ps, Wrs):
            blk = p[...]
            for c in range(blk.shape[1]):
                h = h + jnp.dot(blk[0, c] + blk[1, c], w[:, c * 32:(c + 1) * 32].T,
                                preferred_element_type=jnp.float32, precision=lax.Precision.HIGHEST)
        for u, w in zip(Us, Wts):
            h = h + jnp.dot(u[...], w[...].T, preferred_element_type=jnp.float32, precision=lax.Precision.HIGHEST)
        rid = i * RT + lax.broadcasted_iota(jnp.int32, (RT, 1), 0)
        h = jnp.where(rid < N_NODES, h, 0.0)
        h_ref[...] = h
        s1 = jnp.sum(h, axis=0, keepdims=True)
        s2 = jnp.sum(h * h, axis=0, keepdims=True)
        contrib = jnp.concatenate(
            [s1, s2, jnp.zeros((6, dout), jnp.float32)], axis=0)

        @pl.when(i == 0)
        def _():
            acc[...] = contrib

        @pl.when(i > 0)
        def _():
            acc[...] = acc[...] + contrib

        @pl.when(i == NRT - 1)
        def _():
            st_ref[...] = acc[...]

    in_specs = (
        [pl.BlockSpec((2, p.shape[1], RT, 32), lambda i: (0, 0, i, 0)) for p in P_list]
        + [pl.BlockSpec(w.shape, lambda i: (0, 0)) for w in Wr_list]
        + [pl.BlockSpec((RT, 32), lambda i: (i, 0)) for _ in U_list]
        + [pl.BlockSpec(w.shape, lambda i: (0, 0)) for w in Wt_list]
    )
    return pl.pallas_call(
        body,
        grid=(NRT,),
        in_specs=in_specs,
        out_specs=[pl.BlockSpec((RT, dout), lambda i: (i, 0)),
                   pl.BlockSpec((8, dout), lambda i: (0, 0))],
        out_shape=[jax.ShapeDtypeStruct((NP, dout), jnp.float32),
                   jax.ShapeDtypeStruct((8, dout), jnp.float32)],
        scratch_shapes=[pltpu.VMEM((8, dout), jnp.float32)],
    )(*P_list, *Wr_list, *U_list, *Wt_list)


def _mid_pre(H, svec, ovec, W2r, W2t):
    mid = H.shape[1]

    def body(h_ref, s_ref, o_ref, wr_ref, wt_ref, pre_ref, r2_ref):
        i = pl.program_id(0)
        g = jnp.maximum(h_ref[...] * s_ref[...] + o_ref[...], 0.0)
        rid = i * RT + lax.broadcasted_iota(jnp.int32, (RT, 1), 0)
        g = jnp.where(rid < N_NODES, g, 0.0)
        pre_ref[...] = jnp.dot(g, wr_ref[...].T, preferred_element_type=jnp.float32, precision=lax.Precision.HIGHEST)
        r2_ref[...] = jnp.dot(g, wt_ref[...].T, preferred_element_type=jnp.float32, precision=lax.Precision.HIGHEST)

    return pl.pallas_call(
        body,
        grid=(NRT,),
        in_specs=[pl.BlockSpec((RT, mid), lambda i: (i, 0)),
                  pl.BlockSpec((1, mid), lambda i: (0, 0)),
                  pl.BlockSpec((1, mid), lambda i: (0, 0)),
                  pl.BlockSpec((32, mid), lambda i: (0, 0)),
                  pl.BlockSpec((32, mid), lambda i: (0, 0))],
        out_specs=[pl.BlockSpec((RT, 32), lambda i: (i, 0)),
                   pl.BlockSpec((RT, 32), lambda i: (i, 0))],
        out_shape=[jax.ShapeDtypeStruct((NP, 32), jnp.float32),
                   jax.ShapeDtypeStruct((NP, 32), jnp.float32)],
    )(H, svec, ovec, W2r, W2t)


def _norm_relu(H, svec, ovec):
    C = H.shape[1] // 32

    def body(h_ref, s_ref, o_ref, *outs):
        i = pl.program_id(0)
        g = jnp.maximum(h_ref[...] * s_ref[...] + o_ref[...], 0.0)
        rid = i * RT + lax.broadcasted_iota(jnp.int32, (RT, 1), 0)
        g = jnp.where(rid < N_NODES, g, 0.0)
        for c in range(C):
            outs[c][...] = g[:, c * 32:(c + 1) * 32]

    out = pl.pallas_call(
        body,
        grid=(NRT,),
        in_specs=[pl.BlockSpec((RT, H.shape[1]), lambda i: (i, 0)),
                  pl.BlockSpec((1, H.shape[1]), lambda i: (0, 0)),
                  pl.BlockSpec((1, H.shape[1]), lambda i: (0, 0))],
        out_specs=[pl.BlockSpec((RT, 32), lambda i: (i, 0)) for _ in range(C)],
        out_shape=[jax.ShapeDtypeStruct((NP, 32), jnp.float32) for _ in range(C)],
    )(H, svec, ovec)
    return list(out)


def _pool_lig(h4, batch3, wlig, blig):
    def body(h0_ref, h1_ref, h2_ref, b_ref, w_ref, bb_ref, lig_ref, acc):
        i = pl.program_id(0)
        bt = b_ref[...].reshape(RT, 1)
        oh = (bt == lax.broadcasted_iota(jnp.int32, (RT, N_GRAPHS), 1)
              ).astype(jnp.float32)
        hb = jnp.concatenate(
            [h0_ref[...], h1_ref[...], h2_ref[...], jnp.ones((RT, 32), jnp.float32)],
            axis=1)
        contrib = lax.dot_general(oh, hb, (((0,), (0,)), ((), ())),
                                  preferred_element_type=jnp.float32, precision=lax.Precision.HIGHEST)

        @pl.when(i == 0)
        def _():
            acc[...] = contrib

        @pl.when(i > 0)
        def _():
            acc[...] = acc[...] + contrib

        @pl.when(i == NRT - 1)
        def _():
            cnt = jnp.maximum(acc[:, 96:97], 1.0)
            pooled = acc[:, :96] / cnt
            lig_ref[...] = jnp.dot(pooled, w_ref[...].T,
                                   preferred_element_type=jnp.float32, precision=lax.Precision.HIGHEST) + bb_ref[...]

    return pl.pallas_call(
        body,
        grid=(NRT,),
        in_specs=[pl.BlockSpec((RT, 32), lambda i: (i, 0)),
                  pl.BlockSpec((RT, 32), lambda i: (i, 0)),
                  pl.BlockSpec((RT, 32), lambda i: (i, 0)),
                  pl.BlockSpec((1, 1, RT), lambda i: (i, 0, 0)),
                  pl.BlockSpec((96, 96), lambda i: (0, 0)),
                  pl.BlockSpec((1, 96), lambda i: (0, 0))],
        out_specs=pl.BlockSpec((N_GRAPHS, 96), lambda i: (0, 0)),
        out_shape=jax.ShapeDtypeStruct((N_GRAPHS, 96), jnp.float32),
        scratch_shapes=[pltpu.VMEM((N_GRAPHS, 128), jnp.float32)],
    )(h4[0], h4[1], h4[2], batch3, wlig, blig)


def _prot_conv(target3, embp, wconv):
    def body(*args):
        t_ref, e_ref = args[0], args[1]
        w_refs = args[2:-1]
        out_ref = args[-1]
        tg = t_ref[...].reshape(SEQ, 1)
        oh = (tg == lax.broadcasted_iota(jnp.int32, (SEQ, 32), 1)
              ).astype(jnp.float32)
        e = jnp.dot(oh, e_ref[...], preferred_element_type=jnp.float32, precision=lax.Precision.HIGHEST)
        idx = 0
        maxes = []
        for b in range(3):
            t = e
            for _l in range(b + 1):
                w0, w1, w2, bias = w_refs[idx:idx + 4]
                idx += 4
                lout = t.shape[0] - 2
                y = (jnp.dot(t[0:lout], w0[...].T, preferred_element_type=jnp.float32, precision=lax.Precision.HIGHEST)
                     + jnp.dot(t[1:lout + 1], w1[...].T, preferred_element_type=jnp.float32, precision=lax.Precision.HIGHEST)
                     + jnp.dot(t[2:lout + 2], w2[...].T, preferred_element_type=jnp.float32, precision=lax.Precision.HIGHEST))
                t = jnp.maximum(y + bias[...], 0.0)
            maxes.append(jnp.max(t, axis=0, keepdims=True))
        out_ref[...] = jnp.concatenate(maxes, axis=0)[None]

    in_specs = ([pl.BlockSpec((1, 1, SEQ), lambda g: (g, 0, 0)),
                 pl.BlockSpec((32, 128), lambda g: (0, 0))]
                + [pl.BlockSpec(w.shape, lambda g: (0,) * w.ndim) for w in wconv])
    return pl.pallas_call(
        body,
        grid=(N_GRAPHS,),
        in_specs=in_specs,
        out_specs=pl.BlockSpec((1, 3, 96), lambda g: (g, 0, 0)),
        out_shape=jax.ShapeDtypeStruct((N_GRAPHS, 3, 96), jnp.float32),
    )(target3, embp, *wconv)


def _mlp(pf, lig, wp, bp, w1, b1, w2, b2, w3, b3, w4, b4):
    def body(pf_ref, lig_ref, wp_r, bp_r, w1_r, b1_r, w2_r, b2_r,
             w3_r, b3_r, w4_r, b4_r, out_ref):
        prot = jnp.dot(pf_ref[...], wp_r[...].T,
                       preferred_element_type=jnp.float32, precision=lax.Precision.HIGHEST) + bp_r[...]
        z = jnp.concatenate([prot, lig_ref[...]], axis=1)
        h1 = jnp.maximum(jnp.dot(z, w1_r[...].T,
                                 preferred_element_type=jnp.float32, precision=lax.Precision.HIGHEST) + b1_r[...], 0.0)
        h2 = jnp.maximum(jnp.dot(h1, w2_r[...].T,
                                 preferred_element_type=jnp.float32, precision=lax.Precision.HIGHEST) + b2_r[...], 0.0)
        h3 = jnp.maximum(jnp.dot(h2, w3_r[...].T,
                                 preferred_element_type=jnp.float32, precision=lax.Precision.HIGHEST) + b3_r[...], 0.0)
        out_ref[...] = jnp.dot(h3, w4_r[...].T,
                               preferred_element_type=jnp.float32, precision=lax.Precision.HIGHEST) + b4_r[...]

    return pl.pallas_call(
        body,
        out_shape=jax.ShapeDtypeStruct((N_GRAPHS, 8), jnp.float32),
    )(pf, lig, wp, bp, w1, b1, w2, b2, w3, b3, w4, b4)


def _pad2(a, r, c):
    return jnp.pad(a, ((0, r - a.shape[0]), (0, c - a.shape[1])))


def _split_w(W, widths):
    out = []
    off = 0
    for w in widths:
        out.append(W[:, off:off + w])
        off += w
    return out


def _scale_off(stats, gamma, beta, dout_p):
    m = stats[0:1] / N_NODES
    v = stats[1:2] / N_NODES - m * m
    g = _pad2(gamma.reshape(1, -1), 1, dout_p)
    b = _pad2(beta.reshape(1, -1), 1, dout_p)
    s = g / jnp.sqrt(v + 1e-5)
    return s, b - m * s


def _chunk_w(W, widths, chunk_counts, dout_p):
    whole, chunks = [], []
    for wc, C in zip(_split_w(W, widths), chunk_counts):
        wp = _pad2(wc, dout_p, 32 * C)
        whole.append(wp)
        for c in range(C):
            chunks.append(wp[:, c * 32:(c + 1) * 32])
    return whole, chunks


def kernel(x, edge_index, batch, target, params):
    f32 = jnp.float32
    x32 = jnp.pad(x, ((0, NP - N_NODES), (0, 32 - x.shape[1])))
    srcs = jnp.pad(edge_index[0], (0, EP - N_EDGES),
                   constant_values=NP - 1).reshape(32, NSTEP, ESTEP)
    dsts = jnp.pad(edge_index[1], (0, EP - N_EDGES),
                   constant_values=NP - 1).reshape(32, NSTEP, ESTEP)
    zeros = jnp.zeros((NP, 32), f32)
    batch3 = jnp.pad(batch, (0, NP - N_NODES),
                     constant_values=N_GRAPHS + 8).reshape(NRT, 1, RT)
    target3 = target.reshape(N_GRAPHS, 1, SEQ)

    def agg(chunks):
        return _sc_segsum(len(chunks))(*chunks, srcs, dsts, zeros)

    p0 = params['conv0']
    aggx = agg([x32])
    H, st = _dense_raw([aggx], [x32],
                       [_pad2(p0['rel']['W'], 32, 32)],
                       [_pad2(p0['root']['W'], 32, 32)], 32)
    s, o = _scale_off(st, p0['gamma'], p0['beta'], 32)
    h = _norm_relu(H, s, o)
    feats = [dict(arr=h, agg=agg(h), w=32)]

    eye32 = jnp.eye(32, dtype=f32)
    h4 = None
    for bi in range(4):
        for lp in params['blocks'][bi]:
            mid = lp['c1']['rel']['W'].shape[0]
            widths = [f['w'] for f in feats]
            ccounts = [len(f['arr']) for f in feats]
            wr, _ = _chunk_w(lp['c1']['rel']['W'], widths, ccounts, mid)
            _, wtc = _chunk_w(lp['c1']['root']['W'], widths, ccounts, mid)
            H1, st1 = _dense_raw([f['agg'] for f in feats],
                                 [c for f in feats for c in f['arr']],
                                 wr, wtc, mid)
            s1, o1 = _scale_off(st1, lp['c1']['gamma'], lp['c1']['beta'], mid)
            pre, r2 = _mid_pre(H1, s1, o1, lp['c2']['rel']['W'],
                               lp['c2']['root']['W'])
            P2 = agg([pre])
            traw, st2 = _dense_raw([P2], [r2], [eye32], [eye32], 32)
            s2, o2 = _scale_off(st2, lp['c2']['gamma'], lp['c2']['beta'], 32)
            t = _norm_relu(traw, s2, o2)
            feats.append(dict(arr=t, agg=agg(t), w=32))
        tr = params['trans'][bi]
        dt = tr['rel']['W'].shape[0]
        dp = ((dt + 31) // 32) * 32
        widths = [f['w'] for f in feats]
        ccounts = [len(f['arr']) for f in feats]
        wr, _ = _chunk_w(tr['rel']['W'], widths, ccounts, dp)
        _, wtc = _chunk_w(tr['root']['W'], widths, ccounts, dp)
        Ht, stt = _dense_raw([f['agg'] for f in feats],
                             [c for f in feats for c in f['arr']],
                             wr, wtc, dp)
        st_, ot_ = _scale_off(stt, tr['gamma'], tr['beta'], dp)
        hn = _norm_relu(Ht, st_, ot_)
        if bi < 3:
            feats = [dict(arr=hn, agg=agg(hn), w=dt)]
        else:
            h4 = hn

    lig = _pool_lig(h4, batch3,
                    _pad2(params['lig_cls']['W'], 96, 96),
                    _pad2(params['lig_cls']['b'].reshape(1, -1), 1, 96))

    embp = jnp.pad(params['embed'], ((0, 32 - params['embed'].shape[0]), (0, 0)))
    wconv = []
    for b in range(3):
        for l in range(b + 1):
            cw = params['pblocks'][b][l]
            for k3 in range(3):
                wconv.append(cw['W'][:, :, k3])
            wconv.append(cw['b'].reshape(1, 96))
    pf = _prot_conv(target3, embp, wconv).reshape(N_GRAPHS, 288)

    c = params
    out = _mlp(pf, lig,
               c['plin']['W'], c['plin']['b'].reshape(1, -1),
               c['cls1']['W'], c['cls1']['b'].reshape(1, -1),
               c['cls2']['W'], c['cls2']['b'].reshape(1, -1),
               c['cls3']['W'], c['cls3']['b'].reshape(1, -1),
               _pad2(c['cls4']['W'], 8, 256),
               _pad2(c['cls4']['b'].reshape(1, -1), 1, 8))
    return out[:, :1]


# speedup vs baseline: 1.3932x; 1.3932x over previous
import functools

import jax
import jax.numpy as jnp
from jax import lax
from jax.experimental import pallas as pl
from jax.experimental.pallas import tpu as pltpu
from jax.experimental.pallas import tpu_sc as plsc

N_NODES = 50000
RT = 512
NRT = 98
NP = RT * NRT
N_EDGES = 800000
ESTEP = 256
NSTEP = 98
GSTEP = 14
EP = 32 * NSTEP * ESTEP
RSUB = NP // 16
N_GRAPHS = 256
SEQ = 1000


@functools.cache
def _sc_segsum(C):
    mesh = plsc.VectorSubcoreMesh(core_axis_name="c", subcore_axis_name="s")

    def body(*args):
        vals = args[:C]
        srcs, dsts, zeros, out = args[C:C + 4]
        src_idx, dst_idx, rows, acc, gsem, ssem = args[C + 4:]
        cid = lax.axis_index("c")
        sid = lax.axis_index("s")
        w = cid * 16 + sid

        def gather(c, j, buf):
            pltpu.async_copy(vals[c].at[src_idx.at[j]], rows.at[buf],
                             gsem.at[buf])

        def scat_desc(j, buf):
            return pltpu.make_async_copy(rows.at[buf], acc.at[dst_idx.at[j]],
                                         ssem.at[buf])

        for c in range(C):
            pltpu.sync_copy(zeros.at[pl.ds(sid * RSUB, RSUB)],
                            acc.at[pl.ds(sid * RSUB, RSUB)])
            plsc.subcore_barrier()

            for g in range(NSTEP // GSTEP):
                pltpu.sync_copy(srcs.at[w].at[pl.ds(g * GSTEP, GSTEP)], src_idx)
                pltpu.sync_copy(dsts.at[w].at[pl.ds(g * GSTEP, GSTEP)], dst_idx)
                gather(c, 0, 0)

                def step(j, carry):
                    buf = j % 2
                    pltpu.make_async_copy(vals[c].at[src_idx.at[j]],
                                          rows.at[buf], gsem.at[buf]).wait()

                    @pl.when(j >= 1)
                    def _():
                        scat_desc(j, 1 - buf).wait()

                    @pl.when(j + 1 < GSTEP)
                    def _():
                        gather(c, j + 1, 1 - buf)

                    scat_desc(j, buf).start(add=True)
                    return carry

                lax.fori_loop(0, GSTEP, step, 0)
                scat_desc(0, (GSTEP - 1) % 2).wait()
            plsc.subcore_barrier()
            pltpu.sync_copy(acc.at[pl.ds(sid * RSUB, RSUB)],
                            out.at[cid].at[c].at[pl.ds(sid * RSUB, RSUB)])
            if c + 1 < C:
                plsc.subcore_barrier()

    return pl.kernel(
        body,
        out_type=jax.ShapeDtypeStruct((2, C, NP, 32), jnp.float32),
        mesh=mesh,
        scratch_types=[
            pltpu.VMEM((GSTEP, ESTEP), jnp.int32),
            pltpu.VMEM((GSTEP, ESTEP), jnp.int32),
            pltpu.VMEM((2, ESTEP, 32), jnp.float32),
            pltpu.VMEM_SHARED((NP, 32), jnp.float32),
            pltpu.SemaphoreType.DMA((2,)),
            pltpu.SemaphoreType.DMA((2,)),
        ],
        compiler_params=pltpu.CompilerParams(use_tc_tiling_on_sc=False),
    )


def _dense_raw(P_list, U_list, Wr_list, Wt_list, dout):
    nP, nU = len(P_list), len(U_list)

    def body(*args):
        i = pl.program_id(0)
        it = iter(args)
        Ps = [next(it) for _ in range(nP)]
        Wrs = [next(it) for _ in range(nP)]
        Us = [next(it) for _ in range(nU)]
        Wts = [next(it) for _ in range(nU)]
        h_ref = next(it)
        st_ref = next(it)
        acc = next(it)
        terms = []
        wcols = []
        for p, w in zip(Ps, Wrs):
            blk = p[...]
            for c in range(blk.shape[1]):
                terms.append(blk[0, c] + blk[1, c])
            wcols.append(w[...])
        for u, w in zip(Us, Wts):
            terms.append(u[...])
            wcols.append(w[...])
        big = jnp.concatenate(terms, axis=1) if len(terms) > 1 else terms[0]
        wall = jnp.concatenate(wcols, axis=1) if len(wcols) > 1 else wcols[0]
        h = jnp.dot(big, wall.T, preferred_element_type=jnp.float32)
        rid = i * RT + lax.broadcasted_iota(jnp.int32, (RT, 1), 0)
        h = jnp.where(rid < N_NODES, h, 0.0)
        h_ref[...] = h
        s1 = jnp.sum(h, axis=0, keepdims=True)
        s2 = jnp.sum(h * h, axis=0, keepdims=True)
        contrib = jnp.concatenate(
            [s1, s2, jnp.zeros((6, dout), jnp.float32)], axis=0)

        @pl.when(i == 0)
        def _():
            acc[...] = contrib

        @pl.when(i > 0)
        def _():
            acc[...] = acc[...] + contrib

        @pl.when(i == NRT - 1)
        def _():
            st_ref[...] = acc[...]

    in_specs = (
        [pl.BlockSpec((2, p.shape[1], RT, 32), lambda i: (0, 0, i, 0)) for p in P_list]
        + [pl.BlockSpec(w.shape, lambda i: (0, 0)) for w in Wr_list]
        + [pl.BlockSpec((RT, 32), lambda i: (i, 0)) for _ in U_list]
        + [pl.BlockSpec(w.shape, lambda i: (0, 0)) for w in Wt_list]
    )
    return pl.pallas_call(
        body,
        grid=(NRT,),
        in_specs=in_specs,
        out_specs=[pl.BlockSpec((RT, dout), lambda i: (i, 0)),
                   pl.BlockSpec((8, dout), lambda i: (0, 0))],
        out_shape=[jax.ShapeDtypeStruct((NP, dout), jnp.float32),
                   jax.ShapeDtypeStruct((8, dout), jnp.float32)],
        scratch_shapes=[pltpu.VMEM((8, dout), jnp.float32)],
    )(*P_list, *Wr_list, *U_list, *Wt_list)


def _mid_pre(H, svec, ovec, W2r, W2t):
    mid = H.shape[1]

    def body(h_ref, s_ref, o_ref, wr_ref, wt_ref, pre_ref, r2_ref):
        i = pl.program_id(0)
        g = jnp.maximum(h_ref[...] * s_ref[...] + o_ref[...], 0.0)
        rid = i * RT + lax.broadcasted_iota(jnp.int32, (RT, 1), 0)
        g = jnp.where(rid < N_NODES, g, 0.0)
        pre_ref[...] = jnp.dot(g, wr_ref[...].T, preferred_element_type=jnp.float32)
        r2_ref[...] = jnp.dot(g, wt_ref[...].T, preferred_element_type=jnp.float32)

    return pl.pallas_call(
        body,
        grid=(NRT,),
        in_specs=[pl.BlockSpec((RT, mid), lambda i: (i, 0)),
                  pl.BlockSpec((1, mid), lambda i: (0, 0)),
                  pl.BlockSpec((1, mid), lambda i: (0, 0)),
                  pl.BlockSpec((32, mid), lambda i: (0, 0)),
                  pl.BlockSpec((32, mid), lambda i: (0, 0))],
        out_specs=[pl.BlockSpec((RT, 32), lambda i: (i, 0)),
                   pl.BlockSpec((RT, 32), lambda i: (i, 0))],
        out_shape=[jax.ShapeDtypeStruct((NP, 32), jnp.float32),
                   jax.ShapeDtypeStruct((NP, 32), jnp.float32)],
    )(H, svec, ovec, W2r, W2t)


def _norm_relu(H, svec, ovec):
    C = H.shape[1] // 32

    def body(h_ref, s_ref, o_ref, *outs):
        i = pl.program_id(0)
        g = jnp.maximum(h_ref[...] * s_ref[...] + o_ref[...], 0.0)
        rid = i * RT + lax.broadcasted_iota(jnp.int32, (RT, 1), 0)
        g = jnp.where(rid < N_NODES, g, 0.0)
        for c in range(C):
            outs[c][...] = g[:, c * 32:(c + 1) * 32]

    out = pl.pallas_call(
        body,
        grid=(NRT,),
        in_specs=[pl.BlockSpec((RT, H.shape[1]), lambda i: (i, 0)),
                  pl.BlockSpec((1, H.shape[1]), lambda i: (0, 0)),
                  pl.BlockSpec((1, H.shape[1]), lambda i: (0, 0))],
        out_specs=[pl.BlockSpec((RT, 32), lambda i: (i, 0)) for _ in range(C)],
        out_shape=[jax.ShapeDtypeStruct((NP, 32), jnp.float32) for _ in range(C)],
    )(H, svec, ovec)
    return list(out)


def _pool_lig(h4, batch3, wlig, blig):
    def body(h0_ref, h1_ref, h2_ref, b_ref, w_ref, bb_ref, lig_ref, acc):
        i = pl.program_id(0)
        bt = b_ref[...].reshape(RT, 1)
        oh = (bt == lax.broadcasted_iota(jnp.int32, (RT, N_GRAPHS), 1)
              ).astype(jnp.float32)
        hb = jnp.concatenate(
            [h0_ref[...], h1_ref[...], h2_ref[...], jnp.ones((RT, 32), jnp.float32)],
            axis=1)
        contrib = lax.dot_general(oh, hb, (((0,), (0,)), ((), ())),
                                  preferred_element_type=jnp.float32)

        @pl.when(i == 0)
        def _():
            acc[...] = contrib

        @pl.when(i > 0)
        def _():
            acc[...] = acc[...] + contrib

        @pl.when(i == NRT - 1)
        def _():
            cnt = jnp.maximum(acc[:, 96:97], 1.0)
            pooled = acc[:, :96] / cnt
            lig_ref[...] = jnp.dot(pooled, w_ref[...].T,
                                   preferred_element_type=jnp.float32) + bb_ref[...]

    return pl.pallas_call(
        body,
        grid=(NRT,),
        in_specs=[pl.BlockSpec((RT, 32), lambda i: (i, 0)),
                  pl.BlockSpec((RT, 32), lambda i: (i, 0)),
                  pl.BlockSpec((RT, 32), lambda i: (i, 0)),
                  pl.BlockSpec((1, 1, RT), lambda i: (i, 0, 0)),
                  pl.BlockSpec((96, 96), lambda i: (0, 0)),
                  pl.BlockSpec((1, 96), lambda i: (0, 0))],
        out_specs=pl.BlockSpec((N_GRAPHS, 96), lambda i: (0, 0)),
        out_shape=jax.ShapeDtypeStruct((N_GRAPHS, 96), jnp.float32),
        scratch_shapes=[pltpu.VMEM((N_GRAPHS, 128), jnp.float32)],
    )(h4[0], h4[1], h4[2], batch3, wlig, blig)


def _prot_conv(target3, embp, wconv):
    def body(*args):
        t_ref, e_ref = args[0], args[1]
        w_refs = args[2:-1]
        out_ref = args[-1]
        tg = t_ref[...].reshape(SEQ, 1)
        oh = (tg == lax.broadcasted_iota(jnp.int32, (SEQ, 32), 1)
              ).astype(jnp.float32)
        e = jnp.dot(oh, e_ref[...], preferred_element_type=jnp.float32)
        idx = 0
        maxes = []
        for b in range(3):
            t = e
            for _l in range(b + 1):
                w0, w1, w2, bias = w_refs[idx:idx + 4]
                idx += 4
                lout = t.shape[0] - 2
                y = (jnp.dot(t[0:lout], w0[...].T, preferred_element_type=jnp.float32)
                     + jnp.dot(t[1:lout + 1], w1[...].T, preferred_element_type=jnp.float32)
                     + jnp.dot(t[2:lout + 2], w2[...].T, preferred_element_type=jnp.float32))
                t = jnp.maximum(y + bias[...], 0.0)
            maxes.append(jnp.max(t, axis=0, keepdims=True))
        out_ref[...] = jnp.concatenate(maxes, axis=0)[None]

    in_specs = ([pl.BlockSpec((1, 1, SEQ), lambda g: (g, 0, 0)),
                 pl.BlockSpec((32, 128), lambda g: (0, 0))]
                + [pl.BlockSpec(w.shape, lambda g: (0,) * w.ndim) for w in wconv])
    return pl.pallas_call(
        body,
        grid=(N_GRAPHS,),
        in_specs=in_specs,
        out_specs=pl.BlockSpec((1, 3, 96), lambda g: (g, 0, 0)),
        out_shape=jax.ShapeDtypeStruct((N_GRAPHS, 3, 96), jnp.float32),
    )(target3, embp, *wconv)


def _mlp(pf, lig, wp, bp, w1, b1, w2, b2, w3, b3, w4, b4):
    def body(pf_ref, lig_ref, wp_r, bp_r, w1_r, b1_r, w2_r, b2_r,
             w3_r, b3_r, w4_r, b4_r, out_ref):
        prot = jnp.dot(pf_ref[...], wp_r[...].T,
                       preferred_element_type=jnp.float32) + bp_r[...]
        z = jnp.concatenate([prot, lig_ref[...]], axis=1)
        h1 = jnp.maximum(jnp.dot(z, w1_r[...].T,
                                 preferred_element_type=jnp.float32) + b1_r[...], 0.0)
        h2 = jnp.maximum(jnp.dot(h1, w2_r[...].T,
                                 preferred_element_type=jnp.float32) + b2_r[...], 0.0)
        h3 = jnp.maximum(jnp.dot(h2, w3_r[...].T,
                                 preferred_element_type=jnp.float32) + b3_r[...], 0.0)
        out_ref[...] = jnp.dot(h3, w4_r[...].T,
                               preferred_element_type=jnp.float32) + b4_r[...]

    return pl.pallas_call(
        body,
        out_shape=jax.ShapeDtypeStruct((N_GRAPHS, 8), jnp.float32),
    )(pf, lig, wp, bp, w1, b1, w2, b2, w3, b3, w4, b4)


def _pad2(a, r, c):
    return jnp.pad(a, ((0, r - a.shape[0]), (0, c - a.shape[1])))


def _split_w(W, widths):
    out = []
    off = 0
    for w in widths:
        out.append(W[:, off:off + w])
        off += w
    return out


def _scale_off(stats, gamma, beta, dout_p):
    m = stats[0:1] / N_NODES
    v = stats[1:2] / N_NODES - m * m
    g = _pad2(gamma.reshape(1, -1), 1, dout_p)
    b = _pad2(beta.reshape(1, -1), 1, dout_p)
    s = g / jnp.sqrt(v + 1e-5)
    return s, b - m * s


def _chunk_w(W, widths, chunk_counts, dout_p):
    whole, chunks = [], []
    for wc, C in zip(_split_w(W, widths), chunk_counts):
        wp = _pad2(wc, dout_p, 32 * C)
        whole.append(wp)
        for c in range(C):
            chunks.append(wp[:, c * 32:(c + 1) * 32])
    return whole, chunks


def kernel(x, edge_index, batch, target, params):
    f32 = jnp.float32
    x32 = jnp.pad(x, ((0, NP - N_NODES), (0, 32 - x.shape[1])))
    srcs = jnp.pad(edge_index[0], (0, EP - N_EDGES),
                   constant_values=NP - 1).reshape(32, NSTEP, ESTEP)
    dsts = jnp.pad(edge_index[1], (0, EP - N_EDGES),
                   constant_values=NP - 1).reshape(32, NSTEP, ESTEP)
    zeros = jnp.zeros((NP, 32), f32)
    batch3 = jnp.pad(batch, (0, NP - N_NODES),
                     constant_values=N_GRAPHS + 8).reshape(NRT, 1, RT)
    target3 = target.reshape(N_GRAPHS, 1, SEQ)

    def agg(chunks):
        return _sc_segsum(len(chunks))(*chunks, srcs, dsts, zeros)

    p0 = params['conv0']
    aggx = agg([x32])
    H, st = _dense_raw([aggx], [x32],
                       [_pad2(p0['rel']['W'], 32, 32)],
                       [_pad2(p0['root']['W'], 32, 32)], 32)
    s, o = _scale_off(st, p0['gamma'], p0['beta'], 32)
    h = _norm_relu(H, s, o)
    feats = [dict(arr=h, agg=agg(h), w=32)]

    eye32 = jnp.eye(32, dtype=f32)
    h4 = None
    for bi in range(4):
        for lp in params['blocks'][bi]:
            mid = lp['c1']['rel']['W'].shape[0]
            widths = [f['w'] for f in feats]
            ccounts = [len(f['arr']) for f in feats]
            wr, _ = _chunk_w(lp['c1']['rel']['W'], widths, ccounts, mid)
            _, wtc = _chunk_w(lp['c1']['root']['W'], widths, ccounts, mid)
            H1, st1 = _dense_raw([f['agg'] for f in feats],
                                 [c for f in feats for c in f['arr']],
                                 wr, wtc, mid)
            s1, o1 = _scale_off(st1, lp['c1']['gamma'], lp['c1']['beta'], mid)
            pre, r2 = _mid_pre(H1, s1, o1, lp['c2']['rel']['W'],
                               lp['c2']['root']['W'])
            P2 = agg([pre])
            traw, st2 = _dense_raw([P2], [r2], [eye32], [eye32], 32)
            s2, o2 = _scale_off(st2, lp['c2']['gamma'], lp['c2']['beta'], 32)
            t = _norm_relu(traw, s2, o2)
            feats.append(dict(arr=t, agg=agg(t), w=32))
        tr = params['trans'][bi]
        dt = tr['rel']['W'].shape[0]
        dp = ((dt + 31) // 32) * 32
        widths = [f['w'] for f in feats]
        ccounts = [len(f['arr']) for f in feats]
        wr, _ = _chunk_w(tr['rel']['W'], widths, ccounts, dp)
        _, wtc = _chunk_w(tr['root']['W'], widths, ccounts, dp)
        Ht, stt = _dense_raw([f['agg'] for f in feats],
                             [c for f in feats for c in f['arr']],
                             wr, wtc, dp)
        st_, ot_ = _scale_off(stt, tr['gamma'], tr['beta'], dp)
        hn = _norm_relu(Ht, st_, ot_)
        if bi < 3:
            feats = [dict(arr=hn, agg=agg(hn), w=dt)]
        else:
            h4 = hn

    lig = _pool_lig(h4, batch3,
                    _pad2(params['lig_cls']['W'], 96, 96),
                    _pad2(params['lig_cls']['b'].reshape(1, -1), 1, 96))

    embp = jnp.pad(params['embed'], ((0, 32 - params['embed'].shape[0]), (0, 0)))
    wconv = []
    for b in range(3):
        for l in range(b + 1):
            cw = params['pblocks'][b][l]
            for k3 in range(3):
                wconv.append(cw['W'][:, :, k3])
            wconv.append(cw['b'].reshape(1, 96))
    pf = _prot_conv(target3, embp, wconv).reshape(N_GRAPHS, 288)

    c = params
    out = _mlp(pf, lig,
               c['plin']['W'], c['plin']['b'].reshape(1, -1),
               c['cls1']['W'], c['cls1']['b'].reshape(1, -1),
               c['cls2']['W'], c['cls2']['b'].reshape(1, -1),
               c['cls3']['W'], c['cls3']['b'].reshape(1, -1),
               _pad2(c['cls4']['W'], 8, 256),
               _pad2(c['cls4']['b'].reshape(1, -1), 1, 8))
    return out[:, :1]
